# Initial kernel scaffold; baseline (speedup 1.0000x reference)
import jax
import jax.numpy as jnp
from jax.experimental import pallas as pl


def kernel(feat_src, feat_dst, edge_intra, edge_inter, W1i, al1i, ar1i, b1i, W1e, al1e, ar1e, b1e, W2i, al2i, ar2i, b2i, W2e, al2e, ar2e, b2e, W3i, al3i, ar3i, b3i, W3e, al3e, ar3e, b3e, W4i, al4i, ar4i, b4i, W4e, al4e, ar4e, b4e, lamda1, lamda2, lamda3, lamda4):
    raise NotImplementedError("write your pallas kernel here")


# speedup vs baseline: 20.1332x; 20.1332x over previous
import functools

import jax
import jax.numpy as jnp
from jax import lax
from jax.experimental import pallas as pl
from jax.experimental.pallas import tpu as pltpu
from jax.experimental.pallas import tpu_sc as plsc

N = 10000
D = 128
E = 160000
AW = 144
C = 80
NT = 16
EPT = E // NT
NCH = EPT // C
RPT = N // NT
RZ = 125
NZ = RPT // RZ
NBLK = 10
RB = N // NBLK


def _logits_body(fs_ref, fd_ref, w_ref, al_ref, ar_ref, out_ref):
    rows1, rows2, vrows = [], [], []
    z = jnp.zeros((1, D), jnp.float32)
    for k in range(8):
        u = lax.dot_general(al_ref[k:k + 1, :], w_ref[k],
                            (((1,), (1,)), ((), ())),
                            preferred_element_type=jnp.float32)
        v = lax.dot_general(ar_ref[k:k + 1, :], w_ref[k],
                            (((1,), (1,)), ((), ())),
                            preferred_element_type=jnp.float32)
        if k in (0, 1, 4, 5):
            rows1.append(u)
            rows2.append(z)
        else:
            rows1.append(z)
            rows2.append(u)
        vrows.append(v)
    u1 = jnp.concatenate(rows1 + [jnp.zeros((8, D), jnp.float32)], axis=0)
    u2 = jnp.concatenate(rows2 + vrows, axis=0)
    fs = fs_ref[...]
    fd = fd_ref[...]
    out_ref[...] = (
        lax.dot_general(u1, fs, (((1,), (1,)), ((), ())),
                        preferred_element_type=jnp.float32)
        + lax.dot_general(u2, fd, (((1,), (1,)), ((), ())),
                          preferred_element_type=jnp.float32))


def _sc_edge_body(fsa, fda, edges, eler, zrows, out, acc, el_v, er_v, sidx,
                  didx, exv, rows, sem):
    cid = lax.axis_index("c")
    sid = lax.axis_index("s")

    for g in range(4):
        kd = 2 * g + cid
        table = fsa if g in (0, 2) else fda
        srow = 0 if g < 2 else 1
        drow = 1 - srow

        pltpu.sync_copy(eler.at[pl.ds(kd * N, N)], el_v)
        pltpu.sync_copy(eler.at[pl.ds((8 + kd) * N, N)], er_v)
        for j in range(NZ):
            pltpu.sync_copy(zrows, acc.at[pl.ds(sid * RPT + j * RZ, RZ)])
        plsc.subcore_barrier()

        def chunk(ci, carry):
            off = (cid * 2 + srow) * E + sid * EPT + ci * C
            doff = (cid * 2 + drow) * E + sid * EPT + ci * C
            pltpu.sync_copy(edges.at[pl.ds(off, C)], sidx)
            pltpu.sync_copy(edges.at[pl.ds(doff, C)], didx)
            pltpu.async_copy(table.at[sidx], rows, sem).wait()
            for j in range(C // 16):
                sv = sidx[pl.ds(j * 16, 16)]
                dv = didx[pl.ds(j * 16, 16)]
                e = plsc.load_gather(el_v, [sv]) + plsc.load_gather(er_v, [dv])
                e = jnp.where(e >= 0.0, e, 0.2 * e)
                exv[pl.ds(j * 16, 16)] = jnp.exp(e)

            def scale(ei, c2):
                exb = plsc.load_gather(exv, [jnp.broadcast_to(ei, (16,))])
                for f in range(AW // 16):
                    rows[ei, pl.ds(f * 16, 16)] = (
                        rows[ei, pl.ds(f * 16, 16)] * exb)
                return c2

            lax.fori_loop(0, C, scale, 0)
            pltpu.sync_copy(rows, acc.at[didx], add=True)
            return carry

        lax.fori_loop(0, NCH, chunk, 0)
        plsc.subcore_barrier()

        for j in range(NZ):
            r0 = sid * RPT + j * RZ
            pltpu.sync_copy(acc.at[pl.ds(r0, RZ)], out.at[pl.ds(kd * N + r0, RZ)])


def _combine_body(acc_ref, fd_ref, w_ref, b_ref, lam_ref, out_ref):
    k = pl.program_id(1)
    lamk = lam_ref[k // 2]
    a = acc_ref[0]
    norm = a[:, :D] / (a[:, D:D + 1] + 1e-9)
    rst = lax.dot_general(norm, w_ref[0], (((1,), (0,)), ((), ())),
                          preferred_element_type=jnp.float32)
    contrib = (rst + fd_ref[...] + b_ref[0]) * lamk

    @pl.when(k == 0)
    def _():
        out_ref[...] = contrib

    @pl.when(k != 0)
    def _():
        out_ref[...] += contrib


def kernel(feat_src, feat_dst, edge_intra, edge_inter,
           W1i, al1i, ar1i, b1i, W1e, al1e, ar1e, b1e,
           W2i, al2i, ar2i, b2i, W2e, al2e, ar2e, b2e,
           W3i, al3i, ar3i, b3i, W3e, al3e, ar3e, b3e,
           W4i, al4i, ar4i, b4i, W4e, al4e, ar4e, b4e,
           lamda1, lamda2, lamda3, lamda4):
    f32 = jnp.float32
    ones = jnp.ones((N, 1), f32)
    padz = jnp.zeros((N, AW - D - 1), f32)
    fs_aug = jnp.concatenate([feat_src, ones, padz], axis=1)
    fd_aug = jnp.concatenate([feat_dst, ones, padz], axis=1)
    wstack = jnp.stack([W1i, W1e, W2i, W2e, W3i, W3e, W4i, W4e])
    alstack = jnp.stack([al1i, al1e, al2i, al2e, al3i, al3e, al4i, al4e])
    arstack = jnp.stack([ar1i, ar1e, ar2i, ar2e, ar3i, ar3e, ar4i, ar4e])
    bstack = jnp.stack([b1i, b1e, b2i, b2e, b3i, b3e, b4i, b4e])
    lam = jnp.concatenate([lamda1, lamda2, lamda3, lamda4]).astype(f32)
    edges = jnp.stack([edge_intra, edge_inter]).astype(jnp.int32).reshape(-1)

    eler = pl.pallas_call(
        _logits_body,
        out_shape=jax.ShapeDtypeStruct((16, N), f32),
    )(feat_src, feat_dst, wstack, alstack, arstack).reshape(-1)

    mesh = plsc.VectorSubcoreMesh(core_axis_name="c", subcore_axis_name="s")
    acc = pl.kernel(
        _sc_edge_body,
        mesh=mesh,
        compiler_params=pltpu.CompilerParams(use_tc_tiling_on_sc=False,
                                             needs_layout_passes=False),
        out_type=jax.ShapeDtypeStruct((8 * N, AW), f32),
        scratch_types=[
            pltpu.VMEM_SHARED((N, AW), f32),
            pltpu.VMEM((N,), f32),
            pltpu.VMEM((N,), f32),
            pltpu.VMEM((C,), jnp.int32),
            pltpu.VMEM((C,), jnp.int32),
            pltpu.VMEM((C,), f32),
            pltpu.VMEM((C, AW), f32),
            pltpu.SemaphoreType.DMA,
        ],
    )(fs_aug, fd_aug, edges, eler,
      jnp.zeros((RZ, AW), f32)).reshape(8, N, AW)

    out = pl.pallas_call(
        _combine_body,
        grid=(NBLK, 8),
        in_specs=[
            pl.BlockSpec((1, RB, AW), lambda i, k: (k, i, 0)),
            pl.BlockSpec((RB, D), lambda i, k: (i, 0)),
            pl.BlockSpec((1, D, D), lambda i, k: (k, 0, 0)),
            pl.BlockSpec((1, 1, D), lambda i, k: (k, 0, 0)),
            pl.BlockSpec(memory_space=pltpu.SMEM),
        ],
        out_specs=pl.BlockSpec((RB, D), lambda i, k: (i, 0)),
        out_shape=jax.ShapeDtypeStruct((N, D), f32),
    )(acc, feat_dst, wstack, bstack.reshape(8, 1, D), lam)
    return out


# speedup vs baseline: 22.3436x; 1.1098x over previous
import functools

import jax
import jax.numpy as jnp
from jax import lax
from jax.experimental import pallas as pl
from jax.experimental.pallas import tpu as pltpu
from jax.experimental.pallas import tpu_sc as plsc

N = 10000
D = 128
E = 160000
AW = 144
C = 80
NT = 16
EPT = E // NT
EPTP = 10080
NCH = EPTP // C
HALF = NCH // 2
NP = 10240
SL = NP // NT
RPT = N // NT
RZ = 125
NZ = RPT // RZ
NBLK = 10
RB = N // NBLK


def _logits_body(fs_ref, fd_ref, w_ref, al_ref, ar_ref, out_ref):
    rows1, rows2, vrows = [], [], []
    z = jnp.zeros((1, D), jnp.float32)
    for k in range(8):
        u = lax.dot_general(al_ref[k:k + 1, :], w_ref[k],
                            (((1,), (1,)), ((), ())),
                            preferred_element_type=jnp.float32)
        v = lax.dot_general(ar_ref[k:k + 1, :], w_ref[k],
                            (((1,), (1,)), ((), ())),
                            preferred_element_type=jnp.float32)
        if k in (0, 1, 4, 5):
            rows1.append(u)
            rows2.append(z)
        else:
            rows1.append(z)
            rows2.append(u)
        vrows.append(v)
    u1 = jnp.concatenate(rows1 + [jnp.zeros((8, D), jnp.float32)], axis=0)
    u2 = jnp.concatenate(rows2 + vrows, axis=0)
    fs = fs_ref[...]
    fd = fd_ref[...]
    out_ref[...] = (
        lax.dot_general(u1, fs, (((1,), (1,)), ((), ())),
                        preferred_element_type=jnp.float32)
        + lax.dot_general(u2, fd, (((1,), (1,)), ((), ())),
                          preferred_element_type=jnp.float32))


def _sc_edge_body(fsa, fda, edges, eler, zrows, out, acc, el_sh, er_sh,
                  rows0, rows1, sidx0, sidx1, didx0, didx1,
                  elg0, elg1, erg0, erg1, exv,
                  semr0, semr1, seme0, seme1, semf0, semf1, sems0, sems1):
    cid = lax.axis_index("c")
    sid = lax.axis_index("s")
    rows_b = (rows0, rows1)
    sidx_b = (sidx0, sidx1)
    didx_b = (didx0, didx1)
    elg_b = (elg0, elg1)
    erg_b = (erg0, erg1)
    semr = (semr0, semr1)
    seme = (seme0, seme1)
    semf = (semf0, semf1)
    sems = (sems0, sems1)

    for g in range(4):
        kd = 2 * g + cid
        table = fsa if g in (0, 2) else fda
        srow = 0 if g < 2 else 1
        drow = 1 - srow
        sbase = (cid * 2 + srow) * (NT * EPTP) + sid * EPTP
        dbase = (cid * 2 + drow) * (NT * EPTP) + sid * EPTP

        pltpu.sync_copy(eler.at[pl.ds(kd * NP + sid * SL, SL)],
                        el_sh.at[pl.ds(sid * SL, SL)])
        pltpu.sync_copy(eler.at[pl.ds((8 + kd) * NP + sid * SL, SL)],
                        er_sh.at[pl.ds(sid * SL, SL)])
        for j in range(NZ):
            pltpu.sync_copy(zrows, acc.at[pl.ds(sid * RPT + j * RZ, RZ)])
        plsc.subcore_barrier()

        def issue(b, ci):
            pltpu.sync_copy(edges.at[pl.ds(sbase + ci * C, C)], sidx_b[b])
            pltpu.sync_copy(edges.at[pl.ds(dbase + ci * C, C)], didx_b[b])
            pltpu.async_copy(el_sh.at[sidx_b[b]], elg_b[b], seme[b])
            pltpu.async_copy(er_sh.at[didx_b[b]], erg_b[b], semf[b])
            pltpu.async_copy(table.at[sidx_b[b]], rows_b[b], semr[b])

        def wait_gathers(b):
            pltpu.make_async_copy(el_sh.at[sidx_b[b]], elg_b[b],
                                  seme[b]).wait()
            pltpu.make_async_copy(er_sh.at[didx_b[b]], erg_b[b],
                                  semf[b]).wait()
            pltpu.make_async_copy(table.at[sidx_b[b]], rows_b[b],
                                  semr[b]).wait()

        def compute(b):
            for j in range(C // 16):
                e = (elg_b[b][pl.ds(j * 16, 16)]
                     + erg_b[b][pl.ds(j * 16, 16)])
                e = jnp.where(e >= 0.0, e, 0.2 * e)
                exv[pl.ds(j * 16, 16)] = jnp.exp(e)

            def scale(ei, c2):
                exb = plsc.load_gather(exv, [jnp.broadcast_to(ei, (16,))])
                for f in range(AW // 16):
                    rows_b[b][ei, pl.ds(f * 16, 16)] = (
                        rows_b[b][ei, pl.ds(f * 16, 16)] * exb)
                return c2

            lax.fori_loop(0, C, scale, 0)

        def start_scatter(b):
            pltpu.async_copy(rows_b[b], acc.at[didx_b[b]], sems[b], add=True)

        def wait_scatter(b):
            pltpu.make_async_copy(rows_b[b], acc.at[didx_b[b]],
                                  sems[b]).wait()

        issue(0, 0)

        def pair(p, carry):
            ci0 = 2 * p
            wait_gathers(0)

            @pl.when(p > 0)
            def _():
                wait_scatter(1)

            issue(1, ci0 + 1)
            compute(0)
            start_scatter(0)

            wait_gathers(1)
            wait_scatter(0)

            @pl.when(p < HALF - 1)
            def _():
                issue(0, ci0 + 2)

            compute(1)
            start_scatter(1)
            return carry

        lax.fori_loop(0, HALF, pair, 0)
        wait_scatter(1)
        plsc.subcore_barrier()

        for j in range(NZ):
            r0 = sid * RPT + j * RZ
            pltpu.sync_copy(acc.at[pl.ds(r0, RZ)],
                            out.at[pl.ds(kd * N + r0, RZ)])


def _combine_body(acc_ref, fd_ref, w_ref, b_ref, lam_ref, out_ref):
    k = pl.program_id(1)
    lamk = lam_ref[k // 2]
    a = acc_ref[0]
    norm = a[:, :D] / (a[:, D:D + 1] + 1e-9)
    rst = lax.dot_general(norm, w_ref[0], (((1,), (0,)), ((), ())),
                          preferred_element_type=jnp.float32)
    contrib = (rst + fd_ref[...] + b_ref[0]) * lamk

    @pl.when(k == 0)
    def _():
        out_ref[...] = contrib

    @pl.when(k != 0)
    def _():
        out_ref[...] += contrib


def kernel(feat_src, feat_dst, edge_intra, edge_inter,
           W1i, al1i, ar1i, b1i, W1e, al1e, ar1e, b1e,
           W2i, al2i, ar2i, b2i, W2e, al2e, ar2e, b2e,
           W3i, al3i, ar3i, b3i, W3e, al3e, ar3e, b3e,
           W4i, al4i, ar4i, b4i, W4e, al4e, ar4e, b4e,
           lamda1, lamda2, lamda3, lamda4):
    f32 = jnp.float32
    ones = jnp.ones((N, 1), f32)
    padz = jnp.zeros((N, AW - D - 1), f32)
    fs_aug = jnp.concatenate([feat_src, ones, padz], axis=1)
    fd_aug = jnp.concatenate([feat_dst, ones, padz], axis=1)
    wstack = jnp.stack([W1i, W1e, W2i, W2e, W3i, W3e, W4i, W4e])
    alstack = jnp.stack([al1i, al1e, al2i, al2e, al3i, al3e, al4i, al4e])
    arstack = jnp.stack([ar1i, ar1e, ar2i, ar2e, ar3i, ar3e, ar4i, ar4e])
    bstack = jnp.stack([b1i, b1e, b2i, b2e, b3i, b3e, b4i, b4e])
    lam = jnp.concatenate([lamda1, lamda2, lamda3, lamda4]).astype(f32)
    zpad8 = jnp.zeros((8, AW), f32)
    fs_aug = jnp.concatenate([fs_aug, zpad8], axis=0)
    fd_aug = jnp.concatenate([fd_aug, zpad8], axis=0)
    e4 = jnp.stack([edge_intra[0], edge_intra[1],
                    edge_inter[0], edge_inter[1]]).astype(jnp.int32)
    e4 = e4.reshape(4, NT, EPT)
    e4 = jnp.concatenate(
        [e4, jnp.full((4, NT, EPTP - EPT), N, jnp.int32)], axis=2)
    edges = e4.reshape(-1)

    eler = pl.pallas_call(
        _logits_body,
        out_shape=jax.ShapeDtypeStruct((16, N), f32),
    )(feat_src, feat_dst, wstack, alstack, arstack)
    eler = jnp.pad(eler, ((0, 0), (0, NP - N))).reshape(-1)

    mesh = plsc.VectorSubcoreMesh(core_axis_name="c", subcore_axis_name="s")
    acc = pl.kernel(
        _sc_edge_body,
        mesh=mesh,
        compiler_params=pltpu.CompilerParams(use_tc_tiling_on_sc=False,
                                             needs_layout_passes=False),
        out_type=jax.ShapeDtypeStruct((8 * N, AW), f32),
        scratch_types=[
            pltpu.VMEM_SHARED((N + 8, AW), f32),
            pltpu.VMEM_SHARED((NP,), f32),
            pltpu.VMEM_SHARED((NP,), f32),
            pltpu.VMEM((C, AW), f32),
            pltpu.VMEM((C, AW), f32),
            pltpu.VMEM((C,), jnp.int32),
            pltpu.VMEM((C,), jnp.int32),
            pltpu.VMEM((C,), jnp.int32),
            pltpu.VMEM((C,), jnp.int32),
            pltpu.VMEM((C,), f32),
            pltpu.VMEM((C,), f32),
            pltpu.VMEM((C,), f32),
            pltpu.VMEM((C,), f32),
            pltpu.VMEM((C,), f32),
            pltpu.SemaphoreType.DMA,
            pltpu.SemaphoreType.DMA,
            pltpu.SemaphoreType.DMA,
            pltpu.SemaphoreType.DMA,
            pltpu.SemaphoreType.DMA,
            pltpu.SemaphoreType.DMA,
            pltpu.SemaphoreType.DMA,
            pltpu.SemaphoreType.DMA,
        ],
    )(fs_aug, fd_aug, edges, eler,
      jnp.zeros((RZ, AW), f32)).reshape(8, N, AW)

    out = pl.pallas_call(
        _combine_body,
        grid=(NBLK, 8),
        in_specs=[
            pl.BlockSpec((1, RB, AW), lambda i, k: (k, i, 0)),
            pl.BlockSpec((RB, D), lambda i, k: (i, 0)),
            pl.BlockSpec((1, D, D), lambda i, k: (k, 0, 0)),
            pl.BlockSpec((1, 1, D), lambda i, k: (k, 0, 0)),
            pl.BlockSpec(memory_space=pltpu.SMEM),
        ],
        out_specs=pl.BlockSpec((RB, D), lambda i, k: (i, 0)),
        out_shape=jax.ShapeDtypeStruct((N, D), f32),
    )(acc, feat_dst, wstack, bstack.reshape(8, 1, D), lam)
    return out


# speedup vs baseline: 23.9183x; 1.0705x over previous
import jax
import jax.numpy as jnp
from jax import lax
from jax.experimental import pallas as pl
from jax.experimental.pallas import tpu as pltpu
from jax.experimental.pallas import tpu_sc as plsc

N = 10000
D = 128
E = 160000
C = 96
NT = 16
EPT = E // NT
EPTP = 10176
NCH = EPTP // C
HALF = NCH // 2
NP = 10240
SL = NP // NT
RPT = N // NT
RZ = 125
NZ = RPT // RZ
NBLK = 10
RB = N // NBLK


def _logits_body(fs_ref, fd_ref, w_ref, al_ref, ar_ref, out_ref):
    rows1, rows2, vrows = [], [], []
    z = jnp.zeros((1, D), jnp.float32)
    for k in range(8):
        u = lax.dot_general(al_ref[k:k + 1, :], w_ref[k],
                            (((1,), (1,)), ((), ())),
                            preferred_element_type=jnp.float32)
        v = lax.dot_general(ar_ref[k:k + 1, :], w_ref[k],
                            (((1,), (1,)), ((), ())),
                            preferred_element_type=jnp.float32)
        if k in (0, 1, 4, 5):
            rows1.append(u)
            rows2.append(z)
        else:
            rows1.append(z)
            rows2.append(u)
        vrows.append(v)
    u1 = jnp.concatenate(rows1 + [jnp.zeros((8, D), jnp.float32)], axis=0)
    u2 = jnp.concatenate(rows2 + vrows, axis=0)
    fs = fs_ref[...]
    fd = fd_ref[...]
    out_ref[...] = (
        lax.dot_general(u1, fs, (((1,), (1,)), ((), ())),
                        preferred_element_type=jnp.float32)
        + lax.dot_general(u2, fd, (((1,), (1,)), ((), ())),
                          preferred_element_type=jnp.float32))


def _sc_edge_body(fsa, fda, edges, eler, zrows, zvec, out_num, out_s,
                  acc, s_sh, el_sh, er_sh, sidx, didx,
                  rows0, rows1, elg0, elg1, erg0, erg1, exv0, exv1,
                  semr0, semr1, seme0, seme1, semf0, semf1,
                  sems0, sems1, semx0, semx1):
    cid = lax.axis_index("c")
    sid = lax.axis_index("s")
    rows_b = (rows0, rows1)
    elg_b = (elg0, elg1)
    erg_b = (erg0, erg1)
    exv_b = (exv0, exv1)
    semr = (semr0, semr1)
    seme = (seme0, seme1)
    semf = (semf0, semf1)
    sems = (sems0, sems1)
    semx = (semx0, semx1)

    for g in range(4):
        kd = 2 * g + cid
        table = fsa if g in (0, 2) else fda
        srow = 0 if g < 2 else 1
        drow = 1 - srow
        sbase = (cid * 2 + srow) * (NT * EPTP) + sid * EPTP
        dbase = (cid * 2 + drow) * (NT * EPTP) + sid * EPTP

        pltpu.sync_copy(edges.at[pl.ds(sbase, EPTP)], sidx)
        pltpu.sync_copy(edges.at[pl.ds(dbase, EPTP)], didx)
        pltpu.sync_copy(eler.at[pl.ds(kd * NP + sid * SL, SL)],
                        el_sh.at[pl.ds(sid * SL, SL)])
        pltpu.sync_copy(eler.at[pl.ds((8 + kd) * NP + sid * SL, SL)],
                        er_sh.at[pl.ds(sid * SL, SL)])
        for j in range(NZ):
            pltpu.sync_copy(zrows, acc.at[pl.ds(sid * RPT + j * RZ, RZ)])
        pltpu.sync_copy(zvec, s_sh.at[pl.ds(sid * SL, SL)])

        @pl.when(sid == 0)
        def _():
            pltpu.sync_copy(zrows.at[pl.ds(0, 8)], acc.at[pl.ds(N, 8)])

        plsc.subcore_barrier()

        def issue(b, ci):
            svw = sidx.at[pl.ds(ci * C, C)]
            dvw = didx.at[pl.ds(ci * C, C)]
            pltpu.async_copy(el_sh.at[svw], elg_b[b], seme[b])
            pltpu.async_copy(er_sh.at[dvw], erg_b[b], semf[b])
            pltpu.async_copy(table.at[svw], rows_b[b], semr[b])

        def wait_gathers(b, ci):
            svw = sidx.at[pl.ds(ci * C, C)]
            dvw = didx.at[pl.ds(ci * C, C)]
            pltpu.make_async_copy(el_sh.at[svw], elg_b[b], seme[b]).wait()
            pltpu.make_async_copy(er_sh.at[dvw], erg_b[b], semf[b]).wait()
            pltpu.make_async_copy(table.at[svw], rows_b[b], semr[b]).wait()

        def compute(b):
            for j in range(C // 16):
                e = (elg_b[b][pl.ds(j * 16, 16)]
                     + erg_b[b][pl.ds(j * 16, 16)])
                e = jnp.where(e >= 0.0, e, 0.2 * e)
                exv_b[b][pl.ds(j * 16, 16)] = jnp.exp(e)

            def scale(ei, c2):
                exb = plsc.load_gather(exv_b[b], [jnp.broadcast_to(ei, (16,))])
                for f in range(D // 16):
                    rows_b[b][ei, pl.ds(f * 16, 16)] = (
                        rows_b[b][ei, pl.ds(f * 16, 16)] * exb)
                return c2

            lax.fori_loop(0, C, scale, 0)

        def start_scatter(b, ci):
            dvw = didx.at[pl.ds(ci * C, C)]
            pltpu.async_copy(rows_b[b], acc.at[dvw], sems[b], add=True)
            pltpu.async_copy(exv_b[b], s_sh.at[dvw], semx[b], add=True)

        def wait_scatter(b, ci):
            dvw = didx.at[pl.ds(ci * C, C)]
            pltpu.make_async_copy(rows_b[b], acc.at[dvw], sems[b]).wait()
            pltpu.make_async_copy(exv_b[b], s_sh.at[dvw], semx[b]).wait()

        issue(0, 0)

        def pair(p, carry):
            ci0 = 2 * p
            wait_gathers(0, ci0)

            @pl.when(p > 0)
            def _():
                wait_scatter(1, ci0 - 1)

            issue(1, ci0 + 1)
            compute(0)
            start_scatter(0, ci0)

            wait_gathers(1, ci0 + 1)
            wait_scatter(0, ci0)

            @pl.when(p < HALF - 1)
            def _():
                issue(0, ci0 + 2)

            compute(1)
            start_scatter(1, ci0 + 1)
            return carry

        lax.fori_loop(0, HALF, pair, 0)
        wait_scatter(1, NCH - 1)
        plsc.subcore_barrier()

        for j in range(NZ):
            r0 = sid * RPT + j * RZ
            pltpu.sync_copy(acc.at[pl.ds(r0, RZ)],
                            out_num.at[pl.ds(kd * N + r0, RZ)])
        pltpu.sync_copy(s_sh.at[pl.ds(sid * SL, SL)],
                        out_s.at[pl.ds(kd * NP + sid * SL, SL)])


def _combine_body(acc_ref, s_ref, fd_ref, w_ref, b_ref, lam_ref, out_ref):
    lam = [lam_ref[0], lam_ref[1], lam_ref[2], lam_ref[3]]
    sb = s_ref[...]
    total = fd_ref[...] * (2.0 * (lam[0] + lam[1] + lam[2] + lam[3]))
    for k in range(8):
        inv = 1.0 / (sb[:, k:k + 1] + 1e-9)
        rst = lax.dot_general(acc_ref[k] * inv, w_ref[k],
                              (((1,), (0,)), ((), ())),
                              preferred_element_type=jnp.float32)
        total = total + (rst + b_ref[k:k + 1, :]) * lam[k // 2]
    out_ref[...] = total


def kernel(feat_src, feat_dst, edge_intra, edge_inter,
           W1i, al1i, ar1i, b1i, W1e, al1e, ar1e, b1e,
           W2i, al2i, ar2i, b2i, W2e, al2e, ar2e, b2e,
           W3i, al3i, ar3i, b3i, W3e, al3e, ar3e, b3e,
           W4i, al4i, ar4i, b4i, W4e, al4e, ar4e, b4e,
           lamda1, lamda2, lamda3, lamda4):
    f32 = jnp.float32
    zpad8 = jnp.zeros((8, D), f32)
    fs_pad = jnp.concatenate([feat_src, zpad8], axis=0)
    fd_pad = jnp.concatenate([feat_dst, zpad8], axis=0)
    wstack = jnp.stack([W1i, W1e, W2i, W2e, W3i, W3e, W4i, W4e])
    alstack = jnp.stack([al1i, al1e, al2i, al2e, al3i, al3e, al4i, al4e])
    arstack = jnp.stack([ar1i, ar1e, ar2i, ar2e, ar3i, ar3e, ar4i, ar4e])
    bstack = jnp.stack([b1i, b1e, b2i, b2e, b3i, b3e, b4i, b4e])
    lam = jnp.concatenate([lamda1, lamda2, lamda3, lamda4]).astype(f32)
    e4 = jnp.stack([edge_intra[0], edge_intra[1],
                    edge_inter[0], edge_inter[1]]).astype(jnp.int32)
    e4 = e4.reshape(4, NT, EPT)
    e4 = jnp.concatenate(
        [e4, jnp.full((4, NT, EPTP - EPT), N, jnp.int32)], axis=2)
    edges = e4.reshape(-1)

    eler = pl.pallas_call(
        _logits_body,
        out_shape=jax.ShapeDtypeStruct((16, N), f32),
    )(feat_src, feat_dst, wstack, alstack, arstack)
    eler = jnp.pad(eler, ((0, 0), (0, NP - N))).reshape(-1)

    mesh = plsc.VectorSubcoreMesh(core_axis_name="c", subcore_axis_name="s")
    acc, s_out = pl.kernel(
        _sc_edge_body,
        mesh=mesh,
        compiler_params=pltpu.CompilerParams(use_tc_tiling_on_sc=False,
                                             needs_layout_passes=False),
        out_type=(jax.ShapeDtypeStruct((8 * N, D), f32),
                  jax.ShapeDtypeStruct((8 * NP,), f32)),
        scratch_types=[
            pltpu.VMEM_SHARED((N + 8, D), f32),
            pltpu.VMEM_SHARED((NP,), f32),
            pltpu.VMEM_SHARED((NP,), f32),
            pltpu.VMEM_SHARED((NP,), f32),
            pltpu.VMEM((EPTP,), jnp.int32),
            pltpu.VMEM((EPTP,), jnp.int32),
            pltpu.VMEM((C, D), f32),
            pltpu.VMEM((C, D), f32),
            pltpu.VMEM((C,), f32),
            pltpu.VMEM((C,), f32),
            pltpu.VMEM((C,), f32),
            pltpu.VMEM((C,), f32),
            pltpu.VMEM((C,), f32),
            pltpu.VMEM((C,), f32),
            pltpu.SemaphoreType.DMA,
            pltpu.SemaphoreType.DMA,
            pltpu.SemaphoreType.DMA,
            pltpu.SemaphoreType.DMA,
            pltpu.SemaphoreType.DMA,
            pltpu.SemaphoreType.DMA,
            pltpu.SemaphoreType.DMA,
            pltpu.SemaphoreType.DMA,
            pltpu.SemaphoreType.DMA,
            pltpu.SemaphoreType.DMA,
        ],
    )(fs_pad, fd_pad, edges, eler,
      jnp.zeros((RZ, D), f32), jnp.zeros((SL,), f32))
    acc = acc.reshape(8, N, D)
    s_t = s_out.reshape(8, NP)[:, :N].T

    out = pl.pallas_call(
        _combine_body,
        grid=(NBLK,),
        in_specs=[
            pl.BlockSpec((8, RB, D), lambda i: (0, i, 0)),
            pl.BlockSpec((RB, 8), lambda i: (i, 0)),
            pl.BlockSpec((RB, D), lambda i: (i, 0)),
            pl.BlockSpec((8, D, D), lambda i: (0, 0, 0)),
            pl.BlockSpec((8, D), lambda i: (0, 0)),
            pl.BlockSpec(memory_space=pltpu.SMEM),
        ],
        out_specs=pl.BlockSpec((RB, D), lambda i: (i, 0)),
        out_shape=jax.ShapeDtypeStruct((N, D), f32),
    )(acc, s_t, feat_dst, wstack, bstack, lam)
    return out
